# baseline (speedup 1.0000x reference)
import functools

import jax
import jax.numpy as jnp
from jax import lax
from jax.experimental import pallas as pl
from jax.experimental.pallas import tpu as pltpu
from jax.experimental.pallas import tpu_sc as plsc

N = 10000
E = 320000
D = 128

NC = 2
NS = 16
NW = NC * NS
CB = 64
EP = NW * 128 * 80
PADE = EP - E
EPW = EP // NW
CK = EPW // CB
GC = 8
NG = CK // GC
NP = N + PADE // NW
RPT = NP // NS


def _sc_aggregate_body(x_hbm, ei_hbm, zrows_hbm, zflat_hbm, ones_hbm,
                       y0_hbm, y1_hbm, c0_hbm, c1_hbm,
                       srcv, dstv, rows0, rows1, rows2, rows3, ones,
                       acc, dcnt, s0, s1, s2, s3, semi, semo):
    rows = (rows0, rows1, rows2, rows3)
    sems = (s0, s1, s2, s3)
    cid = lax.axis_index("c")
    sid = lax.axis_index("s")
    wid = sid * NC + cid
    rs = pl.ds(sid * RPT, RPT)

    pltpu.sync_copy(zrows_hbm.at[rs], acc.at[rs])

    @pl.when(sid == 0)
    def _():
        pltpu.sync_copy(zflat_hbm, dcnt)

    pltpu.sync_copy(ones_hbm, ones)

    plsc.subcore_barrier()

    def gather(par, jj, k):
        pltpu.async_copy(x_hbm.at[srcv.at[par, jj]], rows[k], sems[k])

    def wait_gather(par, jj, k):
        pltpu.make_async_copy(x_hbm.at[srcv.at[par, jj]], rows[k], sems[k]).wait()

    def wait_scatter(k):
        pltpu.make_async_copy(rows[k], acc.at[dstv.at[0, 0]], sems[k]).wait()

    pltpu.sync_copy(ei_hbm.at[wid, pl.ds(0, GC)], srcv.at[0])
    pltpu.sync_copy(ei_hbm.at[NW + wid, pl.ds(0, GC)], dstv.at[0])
    gather(0, 0, 0)
    gather(0, 1, 1)

    def group(g, carry):
        par = lax.rem(g, 2)
        nxt = lax.rem(g + 1, 2)
        for jj in range(GC):
            k = jj % 4
            kp = (jj + 2) % 4
            wait_gather(par, jj, k)
            pltpu.async_copy(rows[k], acc.at[dstv.at[par, jj]], sems[k],
                             add=True)
            pltpu.async_copy(ones, dcnt.at[dstv.at[par, jj]], semo, add=True)
            if jj == 2:
                @pl.when(g < NG - 1)
                def _():
                    gs = pl.ds((g + 1) * GC, GC)
                    pltpu.async_copy(ei_hbm.at[wid, gs], srcv.at[nxt], semi)
                    pltpu.async_copy(ei_hbm.at[NW + wid, gs], dstv.at[nxt], semi)
            if jj < GC - 2:
                if jj < 2:
                    @pl.when(g > 0)
                    def _():
                        wait_scatter(kp)
                else:
                    wait_scatter(kp)
                gather(par, jj + 2, kp)
            else:
                @pl.when(g < NG - 1)
                def _():
                    if jj == GC - 2:
                        pltpu.make_async_copy(
                            ei_hbm.at[wid, pl.ds(0, GC)], srcv.at[nxt], semi
                        ).wait()
                        pltpu.make_async_copy(
                            ei_hbm.at[NW + wid, pl.ds(0, GC)], dstv.at[nxt], semi
                        ).wait()
                    wait_scatter(kp)
                    gather(nxt, jj + 2 - GC, kp)

        def drain(p, c2):
            pltpu.make_async_copy(ones, dcnt.at[dstv.at[0, 0]], semo).wait()
            return c2

        lax.fori_loop(0, GC, drain, 0)
        return carry

    lax.fori_loop(0, NG, group, 0)

    for k in range(4):
        wait_scatter(k)

    plsc.subcore_barrier()

    @pl.when(cid == 0)
    def _():
        pltpu.sync_copy(acc.at[rs], y0_hbm.at[rs])

        @pl.when(sid == 0)
        def _():
            pltpu.sync_copy(dcnt, c0_hbm)

    @pl.when(cid == 1)
    def _():
        pltpu.sync_copy(acc.at[rs], y1_hbm.at[rs])

        @pl.when(sid == 0)
        def _():
            pltpu.sync_copy(dcnt, c1_hbm)


@functools.cache
def _sc_aggregate():
    mesh = plsc.VectorSubcoreMesh(core_axis_name="c", subcore_axis_name="s")
    return pl.kernel(
        _sc_aggregate_body,
        out_type=[
            jax.ShapeDtypeStruct((NP, D), jnp.float32),
            jax.ShapeDtypeStruct((NP, D), jnp.float32),
            jax.ShapeDtypeStruct((NP,), jnp.float32),
            jax.ShapeDtypeStruct((NP,), jnp.float32),
        ],
        mesh=mesh,
        scratch_types=[
            pltpu.VMEM((2, GC, CB), jnp.int32),
            pltpu.VMEM((2, GC, CB), jnp.int32),
            pltpu.VMEM((CB, D), jnp.float32),
            pltpu.VMEM((CB, D), jnp.float32),
            pltpu.VMEM((CB, D), jnp.float32),
            pltpu.VMEM((CB, D), jnp.float32),
            pltpu.VMEM((CB,), jnp.float32),
            pltpu.VMEM_SHARED((NP, D), jnp.float32),
            pltpu.VMEM_SHARED((NP,), jnp.float32),
            pltpu.SemaphoreType.DMA,
            pltpu.SemaphoreType.DMA,
            pltpu.SemaphoreType.DMA,
            pltpu.SemaphoreType.DMA,
            pltpu.SemaphoreType.DMA,
            pltpu.SemaphoreType.DMA,
        ],
    )


BN = 2048
BC = BN // D


def _eye():
    r = jax.lax.broadcasted_iota(jnp.int32, (D, D), 0)
    c = jax.lax.broadcasted_iota(jnp.int32, (D, D), 1)
    return (r == c).astype(jnp.float32)


def _col(row_vec):
    return jax.lax.dot_general(
        _eye(), row_vec, (((1,), (1,)), ((), ())),
        preferred_element_type=jnp.float32)


def _deg_cols(c0, c1):
    cs = c0[...] + c1[...]
    cnt = jnp.concatenate([_col(cs[s]) for s in range(BC)], axis=0)
    di = 1.0 / jnp.maximum(cnt, 1.0)
    m = cnt * di
    return di, m


def _mid_body(y0, y1, c0, c1, w1p, b1p, w1t, b1t, w2p, w2t, z):
    di, m = _deg_cols(c0, c1)
    y = (y0[...] + y1[...]) * di
    ap = jnp.dot(y, w1p[...], preferred_element_type=jnp.float32) + m * b1p[...]
    at = jnp.dot(y, w1t[...], preferred_element_type=jnp.float32) + m * b1t[...]
    rp = jnp.maximum(ap, 0.0)
    rt = jnp.maximum(at, 0.0)
    z[...] = (jnp.dot(rt, w2t[...], preferred_element_type=jnp.float32)
              - jnp.dot(rp, w2p[...], preferred_element_type=jnp.float32))


def _loss_body(u0, u1, c0, c1, b2p, b2t, out):
    i = pl.program_id(0)
    nb = pl.num_programs(0)
    di, m = _deg_cols(c0, c1)
    diff = (u0[...] + u1[...]) * di + m * (b2t[...] - b2p[...])
    row = i * BN + jax.lax.broadcasted_iota(jnp.int32, (BN, 1), 0)
    d2 = jnp.where(row < N, diff * diff, 0.0)
    part = jnp.sum(d2)
    tot = jnp.where(i == 0, part, out[...] + part)
    out[...] = tot * jnp.where(i == nb - 1, 1.0 / (N * D), 1.0)


def _row_spec(bn, w):
    return pl.BlockSpec((bn, w), lambda i: (i, 0))


def _full_spec(a, b):
    return pl.BlockSpec((a, b), lambda i: (0, 0))


_tc_mid = pl.pallas_call(
    _mid_body,
    grid=(NP // BN,),
    in_specs=[
        _row_spec(BN, D), _row_spec(BN, D),
        pl.BlockSpec((BC, 1, D), lambda i: (i, 0, 0)),
        pl.BlockSpec((BC, 1, D), lambda i: (i, 0, 0)),
        _full_spec(D, D), _full_spec(1, D),
        _full_spec(D, D), _full_spec(1, D),
        _full_spec(D, D), _full_spec(D, D),
    ],
    out_specs=[_row_spec(BN, D)],
    out_shape=[jax.ShapeDtypeStruct((NP, D), jnp.float32)],
)

_tc_loss = pl.pallas_call(
    _loss_body,
    grid=(NP // BN,),
    in_specs=[
        _row_spec(BN, D), _row_spec(BN, D),
        pl.BlockSpec((BC, 1, D), lambda i: (i, 0, 0)),
        pl.BlockSpec((BC, 1, D), lambda i: (i, 0, 0)),
        _full_spec(1, D), _full_spec(1, D),
    ],
    out_specs=pl.BlockSpec((1, 1), lambda i: (0, 0)),
    out_shape=jax.ShapeDtypeStruct((1, 1), jnp.float32),
)


def kernel(x, edge_index, W1p, b1p, W2p, b2p, W1t, b1t, W2t, b2t):
    npad = PADE // NW
    pad_src = (jnp.arange(npad, dtype=jnp.int32) * 41) % N
    pad_dst = N + jnp.arange(npad, dtype=jnp.int32)
    pad = jnp.broadcast_to(jnp.stack([pad_src, pad_dst])[:, None, :],
                           (2, NW, npad))
    ei3 = jnp.concatenate(
        [edge_index.reshape(2, NW, E // NW), pad], axis=2
    ).reshape(2 * NW, CK, CB)

    xp = jnp.concatenate([x, jnp.zeros((NP - N, D), jnp.float32)])
    zrows = jnp.zeros((NP, D), jnp.float32)
    zflat = jnp.zeros((NP,), jnp.float32)
    ones = jnp.ones((CB,), jnp.float32)

    y0, y1, c0, c1 = _sc_aggregate()(xp, ei3, zrows, zflat, ones)
    c0r = c0.reshape(NP // D, 1, D)
    c1r = c1.reshape(NP // D, 1, D)
    (z,) = _tc_mid(y0, y1, c0r, c1r,
                   W1p, b1p.reshape(1, D), W1t, b1t.reshape(1, D),
                   W2p, W2t)
    u0, u1, _, _ = _sc_aggregate()(z, ei3, zrows, zflat, ones)
    loss = _tc_loss(u0, u1, c0r, c1r, b2p.reshape(1, D), b2t.reshape(1, D))
    return loss.reshape(())


# speedup vs baseline: 1.1179x; 1.1179x over previous
import functools

import jax
import jax.numpy as jnp
from jax import lax
from jax.experimental import pallas as pl
from jax.experimental.pallas import tpu as pltpu
from jax.experimental.pallas import tpu_sc as plsc

N = 10000
E = 320000
D = 128

NC = 2
NS = 16
NW = NC * NS
CB = 128
EP = NW * CB * 80
PADE = EP - E
EPW = EP // NW
CK = EPW // CB
GC = 8
NG = CK // GC
NP = N + PADE // NW
RPT = NP // NS


def _sc_aggregate_body(x_hbm, ei_hbm, zrows_hbm, zflat_hbm, ones_hbm,
                       y0_hbm, y1_hbm, c0_hbm, c1_hbm,
                       srcv, dstv, rows0, rows1, ones,
                       acc, dcnt, s0, s1, semi, semo):
    rows = (rows0, rows1)
    sems = (s0, s1)
    cid = lax.axis_index("c")
    sid = lax.axis_index("s")
    wid = sid * NC + cid
    rs = pl.ds(sid * RPT, RPT)

    pltpu.sync_copy(zrows_hbm.at[rs], acc.at[rs])

    @pl.when(sid == 0)
    def _():
        pltpu.sync_copy(zflat_hbm, dcnt)

    pltpu.sync_copy(ones_hbm, ones)

    plsc.subcore_barrier()

    def gather(par, jj, k):
        pltpu.async_copy(x_hbm.at[srcv.at[par, jj]], rows[k], sems[k])

    def wait_gather(par, jj, k):
        pltpu.make_async_copy(x_hbm.at[srcv.at[par, jj]], rows[k], sems[k]).wait()

    def wait_scatter(k):
        pltpu.make_async_copy(rows[k], acc.at[dstv.at[0, 0]], sems[k]).wait()

    pltpu.sync_copy(ei_hbm.at[wid, pl.ds(0, GC)], srcv.at[0])
    pltpu.sync_copy(ei_hbm.at[NW + wid, pl.ds(0, GC)], dstv.at[0])
    gather(0, 0, 0)
    gather(0, 1, 1)

    def group(g, carry):
        par = lax.rem(g, 2)
        nxt = lax.rem(g + 1, 2)
        for jj in range(GC):
            k = jj % 2
            wait_gather(par, jj, k)
            pltpu.async_copy(rows[k], acc.at[dstv.at[par, jj]], sems[k],
                             add=True)
            pltpu.async_copy(ones, dcnt.at[dstv.at[par, jj]], semo, add=True)
            if jj == 2:
                @pl.when(g < NG - 1)
                def _():
                    gs = pl.ds((g + 1) * GC, GC)
                    pltpu.async_copy(ei_hbm.at[wid, gs], srcv.at[nxt], semi)
                    pltpu.async_copy(ei_hbm.at[NW + wid, gs], dstv.at[nxt], semi)
            wait_scatter(k)
            if jj < GC - 2:
                gather(par, jj + 2, k)
            else:
                @pl.when(g < NG - 1)
                def _():
                    if jj == GC - 2:
                        pltpu.make_async_copy(
                            ei_hbm.at[wid, pl.ds(0, GC)], srcv.at[nxt], semi
                        ).wait()
                        pltpu.make_async_copy(
                            ei_hbm.at[NW + wid, pl.ds(0, GC)], dstv.at[nxt], semi
                        ).wait()
                    gather(nxt, jj + 2 - GC, k)

        def drain(p, c2):
            pltpu.make_async_copy(ones, dcnt.at[dstv.at[0, 0]], semo).wait()
            return c2

        lax.fori_loop(0, GC, drain, 0)
        return carry

    lax.fori_loop(0, NG, group, 0)

    plsc.subcore_barrier()

    @pl.when(cid == 0)
    def _():
        pltpu.sync_copy(acc.at[rs], y0_hbm.at[rs])

        @pl.when(sid == 0)
        def _():
            pltpu.sync_copy(dcnt, c0_hbm)

    @pl.when(cid == 1)
    def _():
        pltpu.sync_copy(acc.at[rs], y1_hbm.at[rs])

        @pl.when(sid == 0)
        def _():
            pltpu.sync_copy(dcnt, c1_hbm)


@functools.cache
def _sc_aggregate():
    mesh = plsc.VectorSubcoreMesh(core_axis_name="c", subcore_axis_name="s")
    return pl.kernel(
        _sc_aggregate_body,
        out_type=[
            jax.ShapeDtypeStruct((NP, D), jnp.float32),
            jax.ShapeDtypeStruct((NP, D), jnp.float32),
            jax.ShapeDtypeStruct((NP,), jnp.float32),
            jax.ShapeDtypeStruct((NP,), jnp.float32),
        ],
        mesh=mesh,
        scratch_types=[
            pltpu.VMEM((2, GC, CB), jnp.int32),
            pltpu.VMEM((2, GC, CB), jnp.int32),
            pltpu.VMEM((CB, D), jnp.float32),
            pltpu.VMEM((CB, D), jnp.float32),
            pltpu.VMEM((CB,), jnp.float32),
            pltpu.VMEM_SHARED((NP, D), jnp.float32),
            pltpu.VMEM_SHARED((NP,), jnp.float32),
            pltpu.SemaphoreType.DMA,
            pltpu.SemaphoreType.DMA,
            pltpu.SemaphoreType.DMA,
            pltpu.SemaphoreType.DMA,
        ],
    )


BN = 2048
BC = BN // D


def _eye():
    r = jax.lax.broadcasted_iota(jnp.int32, (D, D), 0)
    c = jax.lax.broadcasted_iota(jnp.int32, (D, D), 1)
    return (r == c).astype(jnp.float32)


def _col(row_vec):
    return jax.lax.dot_general(
        _eye(), row_vec, (((1,), (1,)), ((), ())),
        preferred_element_type=jnp.float32)


def _deg_cols(c0, c1):
    cs = c0[...] + c1[...]
    cnt = jnp.concatenate([_col(cs[s]) for s in range(BC)], axis=0)
    di = 1.0 / jnp.maximum(cnt, 1.0)
    m = cnt * di
    return di, m


def _mid_body(y0, y1, c0, c1, w1p, b1p, w1t, b1t, w2p, w2t, z):
    di, m = _deg_cols(c0, c1)
    y = (y0[...] + y1[...]) * di
    ap = jnp.dot(y, w1p[...], preferred_element_type=jnp.float32) + m * b1p[...]
    at = jnp.dot(y, w1t[...], preferred_element_type=jnp.float32) + m * b1t[...]
    rp = jnp.maximum(ap, 0.0)
    rt = jnp.maximum(at, 0.0)
    z[...] = (jnp.dot(rt, w2t[...], preferred_element_type=jnp.float32)
              - jnp.dot(rp, w2p[...], preferred_element_type=jnp.float32))


def _loss_body(u0, u1, c0, c1, b2p, b2t, out):
    i = pl.program_id(0)
    nb = pl.num_programs(0)
    di, m = _deg_cols(c0, c1)
    diff = (u0[...] + u1[...]) * di + m * (b2t[...] - b2p[...])
    row = i * BN + jax.lax.broadcasted_iota(jnp.int32, (BN, 1), 0)
    d2 = jnp.where(row < N, diff * diff, 0.0)
    part = jnp.sum(d2)
    tot = jnp.where(i == 0, part, out[...] + part)
    out[...] = tot * jnp.where(i == nb - 1, 1.0 / (N * D), 1.0)


def _row_spec(bn, w):
    return pl.BlockSpec((bn, w), lambda i: (i, 0))


def _full_spec(a, b):
    return pl.BlockSpec((a, b), lambda i: (0, 0))


_tc_mid = pl.pallas_call(
    _mid_body,
    grid=(NP // BN,),
    in_specs=[
        _row_spec(BN, D), _row_spec(BN, D),
        pl.BlockSpec((BC, 1, D), lambda i: (i, 0, 0)),
        pl.BlockSpec((BC, 1, D), lambda i: (i, 0, 0)),
        _full_spec(D, D), _full_spec(1, D),
        _full_spec(D, D), _full_spec(1, D),
        _full_spec(D, D), _full_spec(D, D),
    ],
    out_specs=[_row_spec(BN, D)],
    out_shape=[jax.ShapeDtypeStruct((NP, D), jnp.float32)],
)

_tc_loss = pl.pallas_call(
    _loss_body,
    grid=(NP // BN,),
    in_specs=[
        _row_spec(BN, D), _row_spec(BN, D),
        pl.BlockSpec((BC, 1, D), lambda i: (i, 0, 0)),
        pl.BlockSpec((BC, 1, D), lambda i: (i, 0, 0)),
        _full_spec(1, D), _full_spec(1, D),
    ],
    out_specs=pl.BlockSpec((1, 1), lambda i: (0, 0)),
    out_shape=jax.ShapeDtypeStruct((1, 1), jnp.float32),
)


def kernel(x, edge_index, W1p, b1p, W2p, b2p, W1t, b1t, W2t, b2t):
    npad = PADE // NW
    pad_src = (jnp.arange(npad, dtype=jnp.int32) * 41) % N
    pad_dst = N + jnp.arange(npad, dtype=jnp.int32)
    pad = jnp.broadcast_to(jnp.stack([pad_src, pad_dst])[:, None, :],
                           (2, NW, npad))
    ei3 = jnp.concatenate(
        [edge_index.reshape(2, NW, E // NW), pad], axis=2
    ).reshape(2 * NW, CK, CB)

    xp = jnp.concatenate([x, jnp.zeros((NP - N, D), jnp.float32)])
    zrows = jnp.zeros((NP, D), jnp.float32)
    zflat = jnp.zeros((NP,), jnp.float32)
    ones = jnp.ones((CB,), jnp.float32)

    y0, y1, c0, c1 = _sc_aggregate()(xp, ei3, zrows, zflat, ones)
    c0r = c0.reshape(NP // D, 1, D)
    c1r = c1.reshape(NP // D, 1, D)
    (z,) = _tc_mid(y0, y1, c0r, c1r,
                   W1p, b1p.reshape(1, D), W1t, b1t.reshape(1, D),
                   W2p, W2t)
    u0, u1, _, _ = _sc_aggregate()(z, ei3, zrows, zflat, ones)
    loss = _tc_loss(u0, u1, c0r, c1r, b2p.reshape(1, D), b2t.reshape(1, D))
    return loss.reshape(())
